# baseline (speedup 1.0000x reference)
import functools

import jax
import jax.numpy as jnp
from jax import lax
from jax.experimental import pallas as pl
from jax.experimental.pallas import tpu as pltpu
from jax.experimental.pallas import tpu_sc as plsc

CHUNK = 128


@functools.cache
def _make_lookup(N, D):
    info = plsc.get_sparse_core_info()
    nw = info.num_cores * info.num_subcores
    b_per_w = N // nw
    n_chunks = b_per_w // CHUNK
    mesh = plsc.VectorSubcoreMesh(core_axis_name="c", subcore_axis_name="s")

    @functools.partial(
        pl.kernel,
        mesh=mesh,
        out_type=jax.ShapeDtypeStruct((N, D), jnp.float32),
        compiler_params=pltpu.CompilerParams(use_tc_tiling_on_sc=False),
        scratch_types=[
            pltpu.VMEM((b_per_w,), jnp.int32),
            pltpu.VMEM((CHUNK, D), jnp.float32),
            pltpu.SemaphoreType.DMA,
        ],
    )
    def lookup(idx_hbm, table_hbm, out_hbm, idx_v, rows_v, gsem):
        wid = lax.axis_index("s") * info.num_cores + lax.axis_index("c")
        base = wid * b_per_w
        pltpu.sync_copy(idx_hbm.at[pl.ds(base, b_per_w)], idx_v)

        def body(g, carry):
            pltpu.async_copy(
                table_hbm.at[idx_v.at[pl.ds(g * CHUNK, CHUNK)]], rows_v, gsem
            ).wait()
            pltpu.sync_copy(
                rows_v, out_hbm.at[pl.ds(base + g * CHUNK, CHUNK)]
            )
            return carry

        lax.fori_loop(0, n_chunks, body, 0)

    return lookup


def kernel(x, table):
    B, L = x.shape
    D = table.shape[1]
    idx = x.reshape(-1).astype(jnp.int32)
    out = _make_lookup(B * L, D)(idx, table)
    return out.reshape(B, L, D)


# speedup vs baseline: 1.1155x; 1.1155x over previous
import functools

import jax
import jax.numpy as jnp
from jax import lax
from jax.experimental import pallas as pl
from jax.experimental.pallas import tpu as pltpu
from jax.experimental.pallas import tpu_sc as plsc

CHUNK = 128
K = 4
GROUP = K * CHUNK


@functools.cache
def _make_lookup(N, D):
    info = plsc.get_sparse_core_info()
    nw = info.num_cores * info.num_subcores
    b_per_w = N // nw
    n_chunks = b_per_w // CHUNK
    n_groups = n_chunks // K
    assert n_groups % 2 == 0
    mesh = plsc.VectorSubcoreMesh(core_axis_name="c", subcore_axis_name="s")

    @functools.partial(
        pl.kernel,
        mesh=mesh,
        out_type=jax.ShapeDtypeStruct((N, D), jnp.float32),
        compiler_params=pltpu.CompilerParams(use_tc_tiling_on_sc=False),
        scratch_types=[
            pltpu.VMEM((n_chunks, CHUNK), jnp.int32),
            pltpu.VMEM((GROUP, D), jnp.float32),
            pltpu.VMEM((GROUP, D), jnp.float32),
            pltpu.SemaphoreType.DMA,
            pltpu.SemaphoreType.DMA,
            pltpu.SemaphoreType.DMA,
        ],
    )
    def lookup(idx_hbm, table_hbm, out_hbm, idx_v, buf0, buf1, gsem, osem0,
               osem1):
        wid = lax.axis_index("s") * info.num_cores + lax.axis_index("c")
        base = wid * b_per_w
        pltpu.sync_copy(idx_hbm.at[pl.ds(wid * n_chunks, n_chunks)], idx_v)

        def do_group(g, buf, osem):
            handles = [
                pltpu.async_copy(
                    table_hbm.at[idx_v.at[g * K + j]],
                    buf.at[pl.ds(j * CHUNK, CHUNK)],
                    gsem,
                )
                for j in range(K)
            ]
            for h in handles:
                h.wait()
            pltpu.async_copy(
                buf, out_hbm.at[pl.ds(base + g * GROUP, GROUP)], osem
            )

        def wait_out(g, buf, osem):
            pltpu.make_async_copy(
                buf, out_hbm.at[pl.ds(base + g * GROUP, GROUP)], osem
            ).wait()

        do_group(0, buf0, osem0)
        do_group(1, buf1, osem1)

        def body(i, carry):
            wait_out(2 * i - 2, buf0, osem0)
            do_group(2 * i, buf0, osem0)
            wait_out(2 * i - 1, buf1, osem1)
            do_group(2 * i + 1, buf1, osem1)
            return carry

        lax.fori_loop(1, n_groups // 2, body, 0)
        wait_out(n_groups - 2, buf0, osem0)
        wait_out(n_groups - 1, buf1, osem1)

    return lookup


def kernel(x, table):
    B, L = x.shape
    D = table.shape[1]
    idx = x.reshape(-1, CHUNK).astype(jnp.int32)
    out = _make_lookup(B * L, D)(idx, table)
    return out.reshape(B, L, D)
